# baseline (speedup 1.0000x reference)
import jax
import jax.numpy as jnp
from jax.experimental import pallas as pl
from jax.experimental.pallas import tpu as pltpu


N_TOK = 8192
D_MODEL = 4096
N_GATES = 64
R = 1024
NCHUNK = N_TOK // R
NBUF = 3


def _router_kernel(x_hbm, wt_ref, probs_ref, scores_ref, idx_ref, bufs, sems):
    def copy(c):
        slot = c % NBUF
        return pltpu.make_async_copy(
            x_hbm.at[pl.ds(c * R, R), :], bufs.at[slot], sems.at[slot]
        )

    for c in range(NBUF):
        copy(c).start()

    wt = wt_ref[...]
    for c in range(NCHUNK):
        copy(c).wait()
        logits = jnp.dot(bufs[c % NBUF], wt, preferred_element_type=jnp.float32)
        m = jnp.max(logits, axis=-1, keepdims=True)
        e = jnp.exp(logits - m)
        s = jnp.sum(e, axis=-1, keepdims=True)
        probs = e / s
        probs_ref[pl.ds(c * R, R), :] = probs
        scores_ref[c, :] = 1.0 / s[:, 0]
        idx_ref[c, :] = jnp.argmax(probs, axis=-1).astype(jnp.int32)
        if c + NBUF < NCHUNK:
            copy(c + NBUF).start()


@jax.jit
def kernel(x, W):
    wt = W.T
    probs, scores, idx = pl.pallas_call(
        _router_kernel,
        grid=(),
        in_specs=[
            pl.BlockSpec(memory_space=pltpu.MemorySpace.HBM),
            pl.BlockSpec(memory_space=pltpu.MemorySpace.VMEM),
        ],
        out_specs=[
            pl.BlockSpec(memory_space=pltpu.MemorySpace.VMEM),
            pl.BlockSpec(memory_space=pltpu.MemorySpace.VMEM),
            pl.BlockSpec(memory_space=pltpu.MemorySpace.VMEM),
        ],
        out_shape=[
            jax.ShapeDtypeStruct((N_TOK, N_GATES), jnp.float32),
            jax.ShapeDtypeStruct((NCHUNK, R), jnp.float32),
            jax.ShapeDtypeStruct((NCHUNK, R), jnp.int32),
        ],
        scratch_shapes=[
            pltpu.VMEM((NBUF, R, D_MODEL), jnp.float32),
            pltpu.SemaphoreType.DMA((NBUF,)),
        ],
    )(x, wt)
    return idx.reshape(N_TOK), scores.reshape(N_TOK), probs


# speedup vs baseline: 1.3012x; 1.3012x over previous
import jax
import jax.numpy as jnp
from jax.experimental import pallas as pl
from jax.experimental.pallas import tpu as pltpu


BLK_M = 1024
NSPLIT = 8
SUB_M = BLK_M // NSPLIT


def _router_block(*refs):
    x_refs = refs[:NSPLIT]
    wt_ref, probs_ref, scores_ref, idx_ref = refs[NSPLIT:]
    wt = wt_ref[...]
    for j in range(NSPLIT):
        logits = jnp.dot(x_refs[j][...], wt, preferred_element_type=jnp.float32)
        m = jnp.max(logits, axis=-1, keepdims=True)
        e = jnp.exp(logits - m)
        s = jnp.sum(e, axis=-1, keepdims=True)
        probs = e / s
        probs_ref[pl.ds(j * SUB_M, SUB_M), :] = probs
        scores_ref[0, 0, pl.ds(j * SUB_M, SUB_M)] = 1.0 / s[:, 0]
        idx_ref[0, 0, pl.ds(j * SUB_M, SUB_M)] = jnp.argmax(probs, axis=-1).astype(
            jnp.int32
        )


@jax.jit
def kernel(x, W):
    n_tokens, d_model = x.shape
    n_gates = W.shape[0]
    grid = (n_tokens // BLK_M,)
    wt = W.T

    def x_spec(j):
        return pl.BlockSpec((SUB_M, d_model), lambda i, j=j: (i * NSPLIT + j, 0))

    probs, scores, idx = pl.pallas_call(
        _router_block,
        grid=grid,
        in_specs=[x_spec(j) for j in range(NSPLIT)]
        + [pl.BlockSpec((d_model, n_gates), lambda i: (0, 0))],
        out_specs=[
            pl.BlockSpec((BLK_M, n_gates), lambda i: (i, 0)),
            pl.BlockSpec((1, 1, BLK_M), lambda i: (i, 0, 0)),
            pl.BlockSpec((1, 1, BLK_M), lambda i: (i, 0, 0)),
        ],
        out_shape=[
            jax.ShapeDtypeStruct((n_tokens, n_gates), jnp.float32),
            jax.ShapeDtypeStruct((n_tokens // BLK_M, 1, BLK_M), jnp.float32),
            jax.ShapeDtypeStruct((n_tokens // BLK_M, 1, BLK_M), jnp.int32),
        ],
        compiler_params=pltpu.CompilerParams(
            dimension_semantics=("arbitrary",),
        ),
    )(*([x] * NSPLIT + [wt]))
    return idx.reshape(n_tokens), scores.reshape(n_tokens), probs
